# Initial kernel scaffold; baseline (speedup 1.0000x reference)
import jax
import jax.numpy as jnp
from jax.experimental import pallas as pl


def kernel(x, edge_index_r0, edge_index_r1, edge_index_r2, W_r0, b_r0, W_r1, b_r1, W_r2, b_r2, W_self, b_self):
    raise NotImplementedError("write your pallas kernel here")


# speedup vs baseline: 2.4083x; 2.4083x over previous
import jax
import jax.numpy as jnp
from jax import lax
from jax.experimental import pallas as pl
from jax.experimental.pallas import tpu as pltpu
from jax.experimental.pallas import tpu_sc as plsc

N = 50000
D = 128
E = 200000
NREL = 3

NC, NS, L = 2, 16, 16
NP = 12544
NRANGE = 4
DW = 144
ET = 13312
CH = 1024
NCHUNK = ET // CH
GB = 64
MB = CH + GB
RPT = NP // NS
DUMMY = NP
NPAD = NRANGE * NP


def _sc_body(xa, src3, dst3, zrows, agg, dbuf, sbuf, fsrc, foff, sidx, soff,
             rows, sem, acc):
    c = lax.axis_index("c")
    s = lax.axis_index("s")
    base = s * RPT

    def drain(cnt):
        nb = cnt // GB

        def batch(b, _):
            for j in range(GB // L):
                sidx[pl.ds(j * L, L)] = fsrc[pl.ds(b * GB + j * L, L)]
                soff[pl.ds(j * L, L)] = foff[pl.ds(b * GB + j * L, L)]
            pltpu.async_copy(xa.at[sidx], rows, sem).wait()
            pltpu.sync_copy(rows, acc.at[soff], add=True)
            return 0

        lax.fori_loop(0, nb, batch, 0)
        for j in range(GB // L):
            fsrc[pl.ds(j * L, L)] = fsrc[pl.ds(nb * GB + j * L, L)]
            foff[pl.ds(j * L, L)] = foff[pl.ds(nb * GB + j * L, L)]
        return cnt - nb * GB

    def rng_body(rel, rng):
        rg = 2 * c + rng
        lo = rg * NP

        pltpu.sync_copy(zrows, acc.at[pl.ds(base, RPT)])
        plsc.subcore_barrier()

        def chunk_body(ch, cnt):
            pltpu.sync_copy(dst3.at[rel, s, pl.ds(ch * CH, CH)], dbuf)
            pltpu.sync_copy(src3.at[rel, s, pl.ds(ch * CH, CH)], sbuf)
            for k in range(CH // L):
                dv = dbuf[pl.ds(k * L, L)]
                sv = sbuf[pl.ds(k * L, L)]
                m = (dv >= lo) & (dv < lo + NP)
                mi = m.astype(jnp.int32)
                pos = cnt + jnp.cumsum(mi) - 1
                plsc.store_scatter(foff, [pos], dv - lo, mask=m)
                plsc.store_scatter(fsrc, [pos], sv, mask=m)
                cnt = cnt + jnp.sum(mi)
            return drain(cnt)

        cnt = lax.fori_loop(0, NCHUNK, chunk_body, 0)

        dummyv = jnp.full((L,), DUMMY, jnp.int32)
        zerov = jnp.zeros((L,), jnp.int32)
        for j in range(GB // L):
            foff[pl.ds(cnt + j * L, L)] = dummyv
            fsrc[pl.ds(cnt + j * L, L)] = zerov
        for j in range(GB // L):
            sidx[pl.ds(j * L, L)] = fsrc[pl.ds(j * L, L)]
            soff[pl.ds(j * L, L)] = foff[pl.ds(j * L, L)]
        pltpu.async_copy(xa.at[sidx], rows, sem).wait()
        pltpu.sync_copy(rows, acc.at[soff], add=True)
        plsc.subcore_barrier()

        pltpu.sync_copy(acc.at[pl.ds(base, RPT)],
                        agg.at[rel, rg, pl.ds(base, RPT)])
        plsc.subcore_barrier()
        return 0

    def rel_body(rel, _):
        lax.fori_loop(0, 2, lambda rng, __: rng_body(rel, rng), 0)
        return 0

    lax.fori_loop(0, NREL, rel_body, 0)


def _sc_aggregate(xa, src3, dst3, zrows):
    mesh = plsc.VectorSubcoreMesh(core_axis_name="c", subcore_axis_name="s")
    return pl.kernel(
        _sc_body,
        out_type=jax.ShapeDtypeStruct((NREL, NRANGE, NP, DW), jnp.float32),
        mesh=mesh,
        compiler_params=pltpu.CompilerParams(
            use_tc_tiling_on_sc=False, needs_layout_passes=False),
        scratch_types=[
            pltpu.VMEM((CH,), jnp.int32),
            pltpu.VMEM((CH,), jnp.int32),
            pltpu.VMEM((MB,), jnp.int32),
            pltpu.VMEM((MB,), jnp.int32),
            pltpu.VMEM((GB,), jnp.int32),
            pltpu.VMEM((GB,), jnp.int32),
            pltpu.VMEM((GB, DW), jnp.float32),
            pltpu.SemaphoreType.DMA,
            pltpu.VMEM_SHARED((NP + L, DW), jnp.float32),
        ],
    )(xa, src3, dst3, zrows)


RB = 448
TGRID = NPAD // RB
BPR = NP // RB


def _tc_body(xa_ref, agg_ref, wt_ref, b_ref, out_ref):
    xb = xa_ref[:, :D]
    acc = jnp.dot(xb, wt_ref[0], preferred_element_type=jnp.float32)
    acc += b_ref[0, :][None, :]
    for r in range(NREL):
        ar = agg_ref[r, 0]
        cntc = ar[:, D:D + 1]
        scale = 1.0 / jnp.maximum(cntc, 1.0)
        acc += jnp.dot(ar[:, :D] * scale, wt_ref[r + 1],
                       preferred_element_type=jnp.float32)
        acc += jnp.where(cntc > 0.0, 1.0, 0.0) * b_ref[r + 1, :][None, :]
    out_ref[...] = acc


def _tc_combine(xa, agg, wt_all, b_all):
    return pl.pallas_call(
        _tc_body,
        grid=(TGRID,),
        in_specs=[
            pl.BlockSpec((RB, DW), lambda i: (i, 0)),
            pl.BlockSpec((NREL, 1, RB, DW),
                         lambda i: (0, i // BPR, i % BPR, 0)),
            pl.BlockSpec((NREL + 1, D, D), lambda i: (0, 0, 0)),
            pl.BlockSpec((NREL + 1, D), lambda i: (0, 0)),
        ],
        out_specs=pl.BlockSpec((RB, D), lambda i: (i, 0)),
        out_shape=jax.ShapeDtypeStruct((NPAD, D), jnp.float32),
    )(xa, agg, wt_all, b_all)


def kernel(x, edge_index_r0, edge_index_r1, edge_index_r2, W_r0, b_r0,
           W_r1, b_r1, W_r2, b_r2, W_self, b_self):
    xa = jnp.zeros((NPAD, DW), jnp.float32)
    xa = xa.at[:N, :D].set(x)
    xa = xa.at[:, D].set(1.0)

    pad = NS * ET - E
    srcs, dsts = [], []
    for e in (edge_index_r0, edge_index_r1, edge_index_r2):
        srcs.append(jnp.pad(e[0], (0, pad)).reshape(NS, ET))
        dsts.append(jnp.pad(e[1], (0, pad), constant_values=-1).reshape(NS, ET))
    src3 = jnp.stack(srcs)
    dst3 = jnp.stack(dsts)
    zrows = jnp.zeros((RPT, DW), jnp.float32)

    agg = _sc_aggregate(xa, src3, dst3, zrows)

    wt_all = jnp.stack([W_self.T, W_r0.T, W_r1.T, W_r2.T])
    b_all = jnp.stack([b_self, b_r0, b_r1, b_r2])
    out = _tc_combine(xa, agg, wt_all, b_all)
    return out[:N]
